# Initial kernel scaffold; baseline (speedup 1.0000x reference)
import jax
import jax.numpy as jnp
from jax.experimental import pallas as pl


def kernel(x, edge_index, W1, b1, W2, b2):
    raise NotImplementedError("write your pallas kernel here")


# speedup vs baseline: 7.5160x; 7.5160x over previous
import functools

import jax
import jax.numpy as jnp
from jax import lax
from jax.experimental import pallas as pl
from jax.experimental.pallas import tpu as pltpu
from jax.experimental.pallas import tpu_sc as plsc

N = 10000
E = 160000
D_IN = 256
D_HID = 512
D_OUT = 8

NC = 2
NS = 16
CHUNK = 128
NCHUNK = E // CHUNK
HALF = D_IN // 2
P_W = 16
ZROWS = 200
NZCHUNK = N // ZROWS


def _seg_sum_l1(x0, x1, src, dst, zrows):
    mesh = plsc.VectorSubcoreMesh(core_axis_name="c", subcore_axis_name="s")

    @functools.partial(
        pl.kernel,
        mesh=mesh,
        out_type=[jax.ShapeDtypeStruct((N, HALF), jnp.float32),
                  jax.ShapeDtypeStruct((N, HALF), jnp.float32)],
        scratch_types=[
            pltpu.VMEM((CHUNK,), jnp.int32),
            pltpu.VMEM((CHUNK,), jnp.int32),
            pltpu.VMEM((CHUNK, HALF), jnp.float32),
            pltpu.VMEM_SHARED((N, HALF), jnp.float32),
        ],
    )
    def k(x0_hbm, x1_hbm, src_hbm, dst_hbm, z_hbm, o0_hbm, o1_hbm,
          src_v, dst_v, rows_v, agg_sh):
        cid = lax.axis_index("c")
        sid = lax.axis_index("s")

        def run(x_hbm, o_hbm):
            @pl.loop(sid, NZCHUNK, step=NS)
            def _(j):
                pltpu.sync_copy(z_hbm.at[pl.ds(0, ZROWS)],
                                agg_sh.at[pl.ds(j * ZROWS, ZROWS)])
            plsc.subcore_barrier()

            @pl.loop(sid, NCHUNK, step=NS)
            def _(chunk):
                off = chunk * CHUNK
                pltpu.sync_copy(src_hbm.at[pl.ds(off, CHUNK)], src_v)
                pltpu.sync_copy(dst_hbm.at[pl.ds(off, CHUNK)], dst_v)
                pltpu.sync_copy(x_hbm.at[src_v], rows_v)
                pltpu.sync_copy(rows_v, agg_sh.at[dst_v], add=True)
            plsc.subcore_barrier()

            @pl.loop(sid, NZCHUNK, step=NS)
            def _(j):
                pltpu.sync_copy(agg_sh.at[pl.ds(j * ZROWS, ZROWS)],
                                o_hbm.at[pl.ds(j * ZROWS, ZROWS)])

        @pl.when(cid == 0)
        def _():
            run(x0_hbm, o0_hbm)

        @pl.when(cid == 1)
        def _():
            run(x1_hbm, o1_hbm)

    return k(x0, x1, src, dst, zrows)


def _seg_sum_l2(p, src, dst, zrows):
    mesh = plsc.VectorSubcoreMesh(core_axis_name="c", subcore_axis_name="s")
    half_chunks = NCHUNK // NC

    @functools.partial(
        pl.kernel,
        mesh=mesh,
        compiler_params=pltpu.CompilerParams(use_tc_tiling_on_sc=False),
        out_type=[jax.ShapeDtypeStruct((N, P_W), jnp.float32),
                  jax.ShapeDtypeStruct((N, P_W), jnp.float32)],
        scratch_types=[
            pltpu.VMEM((CHUNK,), jnp.int32),
            pltpu.VMEM((CHUNK,), jnp.int32),
            pltpu.VMEM((CHUNK, P_W), jnp.float32),
            pltpu.VMEM_SHARED((N, P_W), jnp.float32),
        ],
    )
    def k(p_hbm, src_hbm, dst_hbm, z_hbm, oa_hbm, ob_hbm,
          src_v, dst_v, rows_v, agg_sh):
        cid = lax.axis_index("c")
        sid = lax.axis_index("s")

        def run(lo_chunk, o_hbm):
            @pl.loop(sid, NZCHUNK, step=NS)
            def _(j):
                pltpu.sync_copy(z_hbm.at[pl.ds(0, ZROWS)],
                                agg_sh.at[pl.ds(j * ZROWS, ZROWS)])
            plsc.subcore_barrier()

            @pl.loop(lo_chunk + sid, lo_chunk + half_chunks, step=NS)
            def _(chunk):
                off = chunk * CHUNK
                pltpu.sync_copy(src_hbm.at[pl.ds(off, CHUNK)], src_v)
                pltpu.sync_copy(dst_hbm.at[pl.ds(off, CHUNK)], dst_v)
                pltpu.sync_copy(p_hbm.at[src_v], rows_v)
                pltpu.sync_copy(rows_v, agg_sh.at[dst_v], add=True)
            plsc.subcore_barrier()

            @pl.loop(sid, NZCHUNK, step=NS)
            def _(j):
                pltpu.sync_copy(agg_sh.at[pl.ds(j * ZROWS, ZROWS)],
                                o_hbm.at[pl.ds(j * ZROWS, ZROWS)])

        @pl.when(cid == 0)
        def _():
            run(0, oa_hbm)

        @pl.when(cid == 1)
        def _():
            run(half_chunks, ob_hbm)

    return k(p, src, dst, zrows)


_R = 1000


def _tc_a(x, w1_top, b1):
    def body(x_ref, w_ref, b_ref, o_ref):
        o_ref[...] = jnp.dot(x_ref[...], w_ref[...],
                             preferred_element_type=jnp.float32) + b_ref[...]

    return pl.pallas_call(
        body,
        grid=(N // _R,),
        in_specs=[pl.BlockSpec((_R, D_IN), lambda i: (i, 0)),
                  pl.BlockSpec((D_IN, D_HID), lambda i: (0, 0)),
                  pl.BlockSpec((1, D_HID), lambda i: (0, 0))],
        out_specs=pl.BlockSpec((_R, D_HID), lambda i: (i, 0)),
        out_shape=jax.ShapeDtypeStruct((N, D_HID), jnp.float32),
    )(x, w1_top, b1.reshape(1, D_HID))


def _tc_b(u, a0, a1, wb0, wb1, w2_top, w2_bot_pad, b2):
    def body(u_ref, a0_ref, a1_ref, w0_ref, w1_ref, wt_ref, wp_ref, b2_ref,
             t_ref, p_ref):
        h = u_ref[...]
        h = h + jnp.dot(a0_ref[...], w0_ref[...],
                        preferred_element_type=jnp.float32)
        h = h + jnp.dot(a1_ref[...], w1_ref[...],
                        preferred_element_type=jnp.float32)
        h = jnp.maximum(h, 0.0)
        t_ref[...] = jnp.dot(h, wt_ref[...],
                             preferred_element_type=jnp.float32) + b2_ref[...]
        p_ref[...] = jnp.dot(h, wp_ref[...],
                             preferred_element_type=jnp.float32)

    return pl.pallas_call(
        body,
        grid=(N // _R,),
        in_specs=[pl.BlockSpec((_R, D_HID), lambda i: (i, 0)),
                  pl.BlockSpec((_R, HALF), lambda i: (i, 0)),
                  pl.BlockSpec((_R, HALF), lambda i: (i, 0)),
                  pl.BlockSpec((HALF, D_HID), lambda i: (0, 0)),
                  pl.BlockSpec((HALF, D_HID), lambda i: (0, 0)),
                  pl.BlockSpec((D_HID, D_OUT), lambda i: (0, 0)),
                  pl.BlockSpec((D_HID, P_W), lambda i: (0, 0)),
                  pl.BlockSpec((1, D_OUT), lambda i: (0, 0))],
        out_specs=[pl.BlockSpec((_R, D_OUT), lambda i: (i, 0)),
                   pl.BlockSpec((_R, P_W), lambda i: (i, 0))],
        out_shape=[jax.ShapeDtypeStruct((N, D_OUT), jnp.float32),
                   jax.ShapeDtypeStruct((N, P_W), jnp.float32)],
    )(u, a0, a1, wb0, wb1, w2_top, w2_bot_pad, b2.reshape(1, D_OUT))


def _tc_c(t, qa, qb):
    def body(t_ref, a_ref, b_ref, o_ref):
        o_ref[...] = t_ref[...] + a_ref[...] + b_ref[...]

    return pl.pallas_call(
        body,
        grid=(N // _R,),
        in_specs=[pl.BlockSpec((_R, D_OUT), lambda i: (i, 0)),
                  pl.BlockSpec((_R, D_OUT), lambda i: (i, 0)),
                  pl.BlockSpec((_R, D_OUT), lambda i: (i, 0))],
        out_specs=pl.BlockSpec((_R, D_OUT), lambda i: (i, 0)),
        out_shape=jax.ShapeDtypeStruct((N, D_OUT), jnp.float32),
    )(t, qa, qb)


def kernel(x, edge_index, W1, b1, W2, b2):
    src = edge_index[0]
    dst = edge_index[1]
    x0 = x[:, :HALF]
    x1 = x[:, HALF:]
    z1 = jnp.zeros((ZROWS, HALF), jnp.float32)
    z2 = jnp.zeros((ZROWS, P_W), jnp.float32)

    w1_top = W1[:D_IN]
    wb0 = W1[D_IN:D_IN + HALF]
    wb1 = W1[D_IN + HALF:]
    w2_top = W2[:D_HID]
    w2_bot_pad = jnp.concatenate(
        [W2[D_HID:], jnp.zeros((D_HID, P_W - D_OUT), jnp.float32)], axis=1)

    a0, a1 = _seg_sum_l1(x0, x1, src, dst, z1)
    u = _tc_a(x, w1_top, b1)
    t, p = _tc_b(u, a0, a1, wb0, wb1, w2_top, w2_bot_pad, b2)
    pa, pb = _seg_sum_l2(p, src, dst, z2)
    return _tc_c(t, pa[:, :D_OUT], pb[:, :D_OUT])


# speedup vs baseline: 7.8792x; 1.0483x over previous
import functools

import jax
import jax.numpy as jnp
from jax import lax
from jax.experimental import pallas as pl
from jax.experimental.pallas import tpu as pltpu
from jax.experimental.pallas import tpu_sc as plsc

N = 10000
E = 160000
D_IN = 256
D_HID = 512
D_OUT = 8

NC = 2
NS = 16
CHUNK = 128
NBLK = 1280
E_PAD = NBLK * CHUNK
HALF = D_IN // 2
P_W = 16
NROW = N + 8
ZROWS = 200
NZCHUNK = N // ZROWS


def _sc_pipeline(ei_v, x_hbm, agg_sh, rows_a, rows_b, sem_a, sem_b, tpt):
    pltpu.async_copy(x_hbm.at[ei_v.at[0, 0]], rows_a, sem_a)
    pltpu.async_copy(x_hbm.at[ei_v.at[0, 1]], rows_b, sem_b)

    @pl.loop(0, tpt // 2)
    def _(k):
        t = 2 * k
        pltpu.make_async_copy(x_hbm.at[ei_v.at[0, t]], rows_a, sem_a).wait()
        pltpu.sync_copy(rows_a, agg_sh.at[ei_v.at[1, t]], add=True)

        @pl.when(k < tpt // 2 - 1)
        def _():
            pltpu.async_copy(x_hbm.at[ei_v.at[0, t + 2]], rows_a, sem_a)

        pltpu.make_async_copy(x_hbm.at[ei_v.at[0, t + 1]], rows_b,
                              sem_b).wait()
        pltpu.sync_copy(rows_b, agg_sh.at[ei_v.at[1, t + 1]], add=True)

        @pl.when(k < tpt // 2 - 1)
        def _():
            pltpu.async_copy(x_hbm.at[ei_v.at[0, t + 3]], rows_b, sem_b)


def _zero_init(sid, z_hbm, agg_sh):
    @pl.loop(sid, NZCHUNK, step=NS)
    def _(j):
        pltpu.sync_copy(z_hbm.at[pl.ds(0, ZROWS)],
                        agg_sh.at[pl.ds(j * ZROWS, ZROWS)])


def _writeback(sid, agg_sh, o_hbm):
    @pl.loop(sid, NZCHUNK, step=NS)
    def _(j):
        pltpu.sync_copy(agg_sh.at[pl.ds(j * ZROWS, ZROWS)],
                        o_hbm.at[pl.ds(j * ZROWS, ZROWS)])


def _seg_sum_l1(x0, x1, ei3, zrows):
    mesh = plsc.VectorSubcoreMesh(core_axis_name="c", subcore_axis_name="s")
    tpt = NBLK // NS
    halves = 2

    @functools.partial(
        pl.kernel,
        mesh=mesh,
        out_type=[jax.ShapeDtypeStruct((N, HALF), jnp.float32),
                  jax.ShapeDtypeStruct((N, HALF), jnp.float32)],
        scratch_types=[
            pltpu.VMEM((2, NBLK // NS // 2, CHUNK), jnp.int32),
            pltpu.VMEM((CHUNK, HALF), jnp.float32),
            pltpu.VMEM((CHUNK, HALF), jnp.float32),
            pltpu.VMEM_SHARED((NROW, HALF), jnp.float32),
            pltpu.SemaphoreType.DMA,
            pltpu.SemaphoreType.DMA,
        ],
    )
    def k(x0_hbm, x1_hbm, ei_hbm, z_hbm, o0_hbm, o1_hbm,
          ei_v, rows_a, rows_b, agg_sh, sem_a, sem_b):
        cid = lax.axis_index("c")
        sid = lax.axis_index("s")

        def run(x_hbm, o_hbm):
            htp = tpt // halves
            _zero_init(sid, z_hbm, agg_sh)
            plsc.subcore_barrier()
            for h in range(halves):
                pltpu.sync_copy(
                    ei_hbm.at[:, pl.ds(sid * tpt + h * htp, htp), :], ei_v)
                _sc_pipeline(ei_v, x_hbm, agg_sh, rows_a, rows_b,
                             sem_a, sem_b, htp)
            plsc.subcore_barrier()
            _writeback(sid, agg_sh, o_hbm)

        @pl.when(cid == 0)
        def _():
            run(x0_hbm, o0_hbm)

        @pl.when(cid == 1)
        def _():
            run(x1_hbm, o1_hbm)

    return k(x0, x1, ei3, zrows)


def _seg_sum_l2(p, ei3, zrows):
    mesh = plsc.VectorSubcoreMesh(core_axis_name="c", subcore_axis_name="s")
    tpt = NBLK // (NC * NS)

    @functools.partial(
        pl.kernel,
        mesh=mesh,
        compiler_params=pltpu.CompilerParams(use_tc_tiling_on_sc=False),
        out_type=[jax.ShapeDtypeStruct((N, P_W), jnp.float32),
                  jax.ShapeDtypeStruct((N, P_W), jnp.float32)],
        scratch_types=[
            pltpu.VMEM((2, NBLK // (NC * NS), CHUNK), jnp.int32),
            pltpu.VMEM((CHUNK, P_W), jnp.float32),
            pltpu.VMEM((CHUNK, P_W), jnp.float32),
            pltpu.VMEM_SHARED((NROW, P_W), jnp.float32),
            pltpu.SemaphoreType.DMA,
            pltpu.SemaphoreType.DMA,
        ],
    )
    def k(p_hbm, ei_hbm, z_hbm, oa_hbm, ob_hbm,
          ei_v, rows_a, rows_b, agg_sh, sem_a, sem_b):
        cid = lax.axis_index("c")
        sid = lax.axis_index("s")

        def run(lo_chunk, o_hbm):
            pltpu.sync_copy(
                ei_hbm.at[:, pl.ds(lo_chunk + sid * tpt, tpt), :], ei_v)
            _zero_init(sid, z_hbm, agg_sh)
            plsc.subcore_barrier()
            _sc_pipeline(ei_v, p_hbm, agg_sh, rows_a, rows_b,
                         sem_a, sem_b, tpt)
            plsc.subcore_barrier()
            _writeback(sid, agg_sh, o_hbm)

        @pl.when(cid == 0)
        def _():
            run(0, oa_hbm)

        @pl.when(cid == 1)
        def _():
            run(NBLK // NC, ob_hbm)

    return k(p, ei3, zrows)


_R = 1000


def _tc_a(x, w1_top, b1):
    def body(x_ref, w_ref, b_ref, o_ref):
        o_ref[...] = jnp.dot(x_ref[...], w_ref[...],
                             preferred_element_type=jnp.float32) + b_ref[...]

    return pl.pallas_call(
        body,
        grid=(N // _R,),
        in_specs=[pl.BlockSpec((_R, D_IN), lambda i: (i, 0)),
                  pl.BlockSpec((D_IN, D_HID), lambda i: (0, 0)),
                  pl.BlockSpec((1, D_HID), lambda i: (0, 0))],
        out_specs=pl.BlockSpec((_R, D_HID), lambda i: (i, 0)),
        out_shape=jax.ShapeDtypeStruct((N, D_HID), jnp.float32),
    )(x, w1_top, b1.reshape(1, D_HID))


def _tc_b(u, a0, a1, wb0, wb1, w2_top, w2_bot_pad, b2):
    def body(u_ref, a0_ref, a1_ref, w0_ref, w1_ref, wt_ref, wp_ref, b2_ref,
             t_ref, p_ref):
        h = u_ref[...]
        h = h + jnp.dot(a0_ref[...], w0_ref[...],
                        preferred_element_type=jnp.float32)
        h = h + jnp.dot(a1_ref[...], w1_ref[...],
                        preferred_element_type=jnp.float32)
        h = jnp.maximum(h, 0.0)
        t_ref[...] = jnp.dot(h, wt_ref[...],
                             preferred_element_type=jnp.float32) + b2_ref[...]
        p_ref[...] = jnp.dot(h, wp_ref[...],
                             preferred_element_type=jnp.float32)

    return pl.pallas_call(
        body,
        grid=(N // _R,),
        in_specs=[pl.BlockSpec((_R, D_HID), lambda i: (i, 0)),
                  pl.BlockSpec((_R, HALF), lambda i: (i, 0)),
                  pl.BlockSpec((_R, HALF), lambda i: (i, 0)),
                  pl.BlockSpec((HALF, D_HID), lambda i: (0, 0)),
                  pl.BlockSpec((HALF, D_HID), lambda i: (0, 0)),
                  pl.BlockSpec((D_HID, D_OUT), lambda i: (0, 0)),
                  pl.BlockSpec((D_HID, P_W), lambda i: (0, 0)),
                  pl.BlockSpec((1, D_OUT), lambda i: (0, 0))],
        out_specs=[pl.BlockSpec((_R, D_OUT), lambda i: (i, 0)),
                   pl.BlockSpec((_R, P_W), lambda i: (i, 0))],
        out_shape=[jax.ShapeDtypeStruct((N, D_OUT), jnp.float32),
                   jax.ShapeDtypeStruct((N, P_W), jnp.float32)],
    )(u, a0, a1, wb0, wb1, w2_top, w2_bot_pad, b2.reshape(1, D_OUT))


def _tc_c(t, qa, qb):
    def body(t_ref, a_ref, b_ref, o_ref):
        o_ref[...] = t_ref[...] + a_ref[...] + b_ref[...]

    return pl.pallas_call(
        body,
        grid=(N // _R,),
        in_specs=[pl.BlockSpec((_R, D_OUT), lambda i: (i, 0)),
                  pl.BlockSpec((_R, D_OUT), lambda i: (i, 0)),
                  pl.BlockSpec((_R, D_OUT), lambda i: (i, 0))],
        out_specs=pl.BlockSpec((_R, D_OUT), lambda i: (i, 0)),
        out_shape=jax.ShapeDtypeStruct((N, D_OUT), jnp.float32),
    )(t, qa, qb)


def kernel(x, edge_index, W1, b1, W2, b2):
    src = jnp.concatenate([edge_index[0],
                           jnp.zeros((E_PAD - E,), jnp.int32)])
    dst = jnp.concatenate([edge_index[1],
                           jnp.full((E_PAD - E,), N, jnp.int32)])
    ei3 = jnp.stack([src, dst]).reshape(2, NBLK, CHUNK)
    x0 = x[:, :HALF]
    x1 = x[:, HALF:]
    z1 = jnp.zeros((ZROWS, HALF), jnp.float32)
    z2 = jnp.zeros((ZROWS, P_W), jnp.float32)

    w1_top = W1[:D_IN]
    wb0 = W1[D_IN:D_IN + HALF]
    wb1 = W1[D_IN + HALF:]
    w2_top = W2[:D_HID]
    w2_bot_pad = jnp.concatenate(
        [W2[D_HID:], jnp.zeros((D_HID, P_W - D_OUT), jnp.float32)], axis=1)

    a0, a1 = _seg_sum_l1(x0, x1, ei3, z1)
    u = _tc_a(x, w1_top, b1)
    t, p = _tc_b(u, a0, a1, wb0, wb1, w2_top, w2_bot_pad, b2)
    pa, pb = _seg_sum_l2(p, ei3, z2)
    return _tc_c(t, pa[:, :D_OUT], pb[:, :D_OUT])


# speedup vs baseline: 9.3158x; 1.1823x over previous
import functools

import jax
import jax.numpy as jnp
from jax import lax
from jax.experimental import pallas as pl
from jax.experimental.pallas import tpu as pltpu
from jax.experimental.pallas import tpu_sc as plsc

N = 10000
E = 160000
D_IN = 256
D_HID = 512
D_OUT = 8

NC = 2
NS = 16
CHUNK = 128
NBLK = 1280
E_PAD = NBLK * CHUNK
QW = D_IN // 4
P_W = 16
NROW = N + 8
ZROWS = 200
NZCHUNK = N // ZROWS
NBUF = 4
IDXH = 40


def _zero_init(sid, z_hbm, agg_sh):
    @pl.loop(sid, NZCHUNK, step=NS)
    def _(j):
        pltpu.sync_copy(z_hbm.at[pl.ds(0, ZROWS)],
                        agg_sh.at[pl.ds(j * ZROWS, ZROWS)])


def _writeback(sid, agg_sh, o_hbm):
    @pl.loop(sid, NZCHUNK, step=NS)
    def _(j):
        pltpu.sync_copy(agg_sh.at[pl.ds(j * ZROWS, ZROWS)],
                        o_hbm.at[pl.ds(j * ZROWS, ZROWS)])


def _seg_sum_l1(xq, ei3, zrows):
    mesh = plsc.VectorSubcoreMesh(core_axis_name="c", subcore_axis_name="s")
    tpt = NBLK // NS

    @functools.partial(
        pl.kernel,
        mesh=mesh,
        compiler_params=pltpu.CompilerParams(use_tc_tiling_on_sc=False),
        out_type=[jax.ShapeDtypeStruct((N, QW), jnp.float32)
                  for _ in range(4)],
        scratch_types=[
            pltpu.VMEM((2, IDXH, CHUNK), jnp.int32),
            pltpu.VMEM((NBUF, CHUNK, QW), jnp.float32),
            pltpu.VMEM_SHARED((N, QW), jnp.float32),
            pltpu.VMEM_SHARED((NROW, QW), jnp.float32),
        ] + [pltpu.SemaphoreType.DMA for _ in range(NBUF)],
    )
    def k(xq_hbm, ei_hbm, z_hbm, o0_hbm, o1_hbm, o2_hbm, o3_hbm,
          ei_v, rows_v, x_sh, agg_sh, *sems):
        cid = lax.axis_index("c")
        sid = lax.axis_index("s")

        def one_pass(q, o_hbm):
            @pl.loop(sid, NZCHUNK, step=NS)
            def _(j):
                pltpu.sync_copy(xq_hbm.at[q, pl.ds(j * ZROWS, ZROWS), :],
                                x_sh.at[pl.ds(j * ZROWS, ZROWS)])
                pltpu.sync_copy(z_hbm.at[pl.ds(0, ZROWS)],
                                agg_sh.at[pl.ds(j * ZROWS, ZROWS)])
            plsc.subcore_barrier()

            for h in range(tpt // IDXH):
                pltpu.sync_copy(
                    ei_hbm.at[:, pl.ds(sid * tpt + h * IDXH, IDXH), :], ei_v)
                for b in range(NBUF):
                    pltpu.async_copy(x_sh.at[ei_v.at[0, b]],
                                     rows_v.at[b], sems[b])

                @pl.loop(0, IDXH // NBUF)
                def _(kk):
                    for b in range(NBUF):
                        t = NBUF * kk + b
                        pltpu.make_async_copy(x_sh.at[ei_v.at[0, t]],
                                              rows_v.at[b], sems[b]).wait()
                        pltpu.sync_copy(rows_v.at[b],
                                        agg_sh.at[ei_v.at[1, t]], add=True)

                        @pl.when(kk < IDXH // NBUF - 1)
                        def _():
                            pltpu.async_copy(x_sh.at[ei_v.at[0, t + NBUF]],
                                             rows_v.at[b], sems[b])
            plsc.subcore_barrier()
            _writeback(sid, agg_sh, o_hbm)
            plsc.subcore_barrier()

        @pl.when(cid == 0)
        def _():
            one_pass(0, o0_hbm)
            one_pass(1, o1_hbm)

        @pl.when(cid == 1)
        def _():
            one_pass(2, o2_hbm)
            one_pass(3, o3_hbm)

    return k(xq, ei3, zrows)


def _seg_sum_l2(p, ei3, zrows):
    mesh = plsc.VectorSubcoreMesh(core_axis_name="c", subcore_axis_name="s")
    tpt = NBLK // (NC * NS)

    @functools.partial(
        pl.kernel,
        mesh=mesh,
        compiler_params=pltpu.CompilerParams(use_tc_tiling_on_sc=False),
        out_type=[jax.ShapeDtypeStruct((N, P_W), jnp.float32),
                  jax.ShapeDtypeStruct((N, P_W), jnp.float32)],
        scratch_types=[
            pltpu.VMEM((2, NBLK // (NC * NS), CHUNK), jnp.int32),
            pltpu.VMEM((NBUF, CHUNK, P_W), jnp.float32),
            pltpu.VMEM_SHARED((NROW, P_W), jnp.float32),
        ] + [pltpu.SemaphoreType.DMA for _ in range(NBUF)],
    )
    def k(p_hbm, ei_hbm, z_hbm, oa_hbm, ob_hbm,
          ei_v, rows_v, agg_sh, *sems):
        cid = lax.axis_index("c")
        sid = lax.axis_index("s")

        def run(lo_chunk, o_hbm):
            pltpu.sync_copy(
                ei_hbm.at[:, pl.ds(lo_chunk + sid * tpt, tpt), :], ei_v)
            _zero_init(sid, z_hbm, agg_sh)
            plsc.subcore_barrier()
            for b in range(NBUF):
                pltpu.async_copy(p_hbm.at[ei_v.at[0, b]], rows_v.at[b],
                                 sems[b])

            @pl.loop(0, tpt // NBUF)
            def _(kk):
                for b in range(NBUF):
                    t = NBUF * kk + b
                    pltpu.make_async_copy(p_hbm.at[ei_v.at[0, t]],
                                          rows_v.at[b], sems[b]).wait()
                    pltpu.sync_copy(rows_v.at[b],
                                    agg_sh.at[ei_v.at[1, t]], add=True)

                    @pl.when(kk < tpt // NBUF - 1)
                    def _():
                        pltpu.async_copy(p_hbm.at[ei_v.at[0, t + NBUF]],
                                         rows_v.at[b], sems[b])
            plsc.subcore_barrier()
            _writeback(sid, agg_sh, o_hbm)

        @pl.when(cid == 0)
        def _():
            run(0, oa_hbm)

        @pl.when(cid == 1)
        def _():
            run(NBLK // NC, ob_hbm)

    return k(p, ei3, zrows)


_R = 1000


def _tc_a(x, w1_top, b1):
    def body(x_ref, w_ref, b_ref, o_ref):
        o_ref[...] = jnp.dot(x_ref[...], w_ref[...],
                             preferred_element_type=jnp.float32) + b_ref[...]

    return pl.pallas_call(
        body,
        grid=(N // _R,),
        in_specs=[pl.BlockSpec((_R, D_IN), lambda i: (i, 0)),
                  pl.BlockSpec((D_IN, D_HID), lambda i: (0, 0)),
                  pl.BlockSpec((1, D_HID), lambda i: (0, 0))],
        out_specs=pl.BlockSpec((_R, D_HID), lambda i: (i, 0)),
        out_shape=jax.ShapeDtypeStruct((N, D_HID), jnp.float32),
    )(x, w1_top, b1.reshape(1, D_HID))


def _tc_b(u, aggs, w1_bot, w2_top, w2_bot_pad, b2):
    def body(u_ref, a0_ref, a1_ref, a2_ref, a3_ref, wb_ref, wt_ref, wp_ref,
             b2_ref, t_ref, p_ref):
        agg = jnp.concatenate(
            [a0_ref[...], a1_ref[...], a2_ref[...], a3_ref[...]], axis=1)
        h = u_ref[...] + jnp.dot(agg, wb_ref[...],
                                 preferred_element_type=jnp.float32)
        h = jnp.maximum(h, 0.0)
        t_ref[...] = jnp.dot(h, wt_ref[...],
                             preferred_element_type=jnp.float32) + b2_ref[...]
        p_ref[...] = jnp.dot(h, wp_ref[...],
                             preferred_element_type=jnp.float32)

    return pl.pallas_call(
        body,
        grid=(N // _R,),
        in_specs=[pl.BlockSpec((_R, D_HID), lambda i: (i, 0))] +
                 [pl.BlockSpec((_R, QW), lambda i: (i, 0))
                  for _ in range(4)] +
                 [pl.BlockSpec((D_IN, D_HID), lambda i: (0, 0)),
                  pl.BlockSpec((D_HID, D_OUT), lambda i: (0, 0)),
                  pl.BlockSpec((D_HID, P_W), lambda i: (0, 0)),
                  pl.BlockSpec((1, D_OUT), lambda i: (0, 0))],
        out_specs=[pl.BlockSpec((_R, D_OUT), lambda i: (i, 0)),
                   pl.BlockSpec((_R, P_W), lambda i: (i, 0))],
        out_shape=[jax.ShapeDtypeStruct((N, D_OUT), jnp.float32),
                   jax.ShapeDtypeStruct((N, P_W), jnp.float32)],
    )(u, *aggs, w1_bot, w2_top, w2_bot_pad, b2.reshape(1, D_OUT))


def _tc_c(t, qa, qb):
    def body(t_ref, a_ref, b_ref, o_ref):
        o_ref[...] = t_ref[...] + a_ref[...] + b_ref[...]

    return pl.pallas_call(
        body,
        grid=(N // _R,),
        in_specs=[pl.BlockSpec((_R, D_OUT), lambda i: (i, 0)),
                  pl.BlockSpec((_R, D_OUT), lambda i: (i, 0)),
                  pl.BlockSpec((_R, D_OUT), lambda i: (i, 0))],
        out_specs=pl.BlockSpec((_R, D_OUT), lambda i: (i, 0)),
        out_shape=jax.ShapeDtypeStruct((N, D_OUT), jnp.float32),
    )(t, qa, qb)


def kernel(x, edge_index, W1, b1, W2, b2):
    src = jnp.concatenate([edge_index[0],
                           jnp.zeros((E_PAD - E,), jnp.int32)])
    dst = jnp.concatenate([edge_index[1],
                           jnp.full((E_PAD - E,), N, jnp.int32)])
    ei3 = jnp.stack([src, dst]).reshape(2, NBLK, CHUNK)
    xq = x.reshape(N, 4, QW).transpose(1, 0, 2)
    z1 = jnp.zeros((ZROWS, QW), jnp.float32)
    z2 = jnp.zeros((ZROWS, P_W), jnp.float32)

    w1_top = W1[:D_IN]
    w1_bot = W1[D_IN:]
    w2_top = W2[:D_HID]
    w2_bot_pad = jnp.concatenate(
        [W2[D_HID:], jnp.zeros((D_HID, P_W - D_OUT), jnp.float32)], axis=1)

    aggs = _seg_sum_l1(xq, ei3, z1)
    u = _tc_a(x, w1_top, b1)
    t, p = _tc_b(u, aggs, w1_bot, w2_top, w2_bot_pad, b2)
    pa, pb = _seg_sum_l2(p, ei3, z2)
    return _tc_c(t, pa[:, :D_OUT], pb[:, :D_OUT])


# speedup vs baseline: 10.6841x; 1.1469x over previous
import functools

import jax
import jax.numpy as jnp
from jax import lax
from jax.experimental import pallas as pl
from jax.experimental.pallas import tpu as pltpu
from jax.experimental.pallas import tpu_sc as plsc

N = 10000
E = 160000
D_IN = 256
D_HID = 512
D_OUT = 8

NC = 2
NS = 16
CHUNK = 128
NBLK = 1280
E_PAD = NBLK * CHUNK
QW = D_IN // 4
P_W = 16
NROW = N + 8
ZROWS = 200
NZCHUNK = N // ZROWS
NBUF = 4
IDXH = 40


def _zero_init(sid, z_hbm, agg_sh):
    @pl.loop(sid, NZCHUNK, step=NS)
    def _(j):
        pltpu.sync_copy(z_hbm.at[pl.ds(0, ZROWS)],
                        agg_sh.at[pl.ds(j * ZROWS, ZROWS)])


def _writeback(sid, agg_sh, o_hbm):
    @pl.loop(sid, NZCHUNK, step=NS)
    def _(j):
        pltpu.sync_copy(agg_sh.at[pl.ds(j * ZROWS, ZROWS)],
                        o_hbm.at[pl.ds(j * ZROWS, ZROWS)])


def _seg_sum_l1(x, ei3, zrows):
    mesh = plsc.VectorSubcoreMesh(core_axis_name="c", subcore_axis_name="s")
    tpt = NBLK // NS

    @functools.partial(
        pl.kernel,
        mesh=mesh,
        compiler_params=pltpu.CompilerParams(use_tc_tiling_on_sc=False),
        out_type=[jax.ShapeDtypeStruct((N, QW), jnp.float32)
                  for _ in range(4)],
        scratch_types=[
            pltpu.VMEM((2, IDXH, CHUNK), jnp.int32),
            pltpu.VMEM((NBUF, CHUNK, QW), jnp.float32),
            pltpu.VMEM_SHARED((N, QW), jnp.float32),
            pltpu.VMEM_SHARED((NROW, QW), jnp.float32),
        ] + [pltpu.SemaphoreType.DMA for _ in range(NBUF)],
    )
    def k(x_hbm, ei_hbm, z_hbm, o0_hbm, o1_hbm, o2_hbm, o3_hbm,
          ei_v, rows_v, x_sh, agg_sh, *sems):
        cid = lax.axis_index("c")
        sid = lax.axis_index("s")

        def one_pass(q, o_hbm):
            @pl.loop(sid, NZCHUNK, step=NS)
            def _(j):
                pltpu.sync_copy(
                    x_hbm.at[pl.ds(j * ZROWS, ZROWS), pl.ds(q * QW, QW)],
                    x_sh.at[pl.ds(j * ZROWS, ZROWS)])
                pltpu.sync_copy(z_hbm.at[pl.ds(0, ZROWS)],
                                agg_sh.at[pl.ds(j * ZROWS, ZROWS)])
            plsc.subcore_barrier()

            for h in range(tpt // IDXH):
                pltpu.sync_copy(
                    ei_hbm.at[:, pl.ds(sid * tpt + h * IDXH, IDXH), :], ei_v)
                for b in range(NBUF):
                    pltpu.async_copy(x_sh.at[ei_v.at[0, b]],
                                     rows_v.at[b], sems[b])

                @pl.loop(0, IDXH // NBUF)
                def _(kk):
                    for b in range(NBUF):
                        t = NBUF * kk + b
                        pltpu.make_async_copy(x_sh.at[ei_v.at[0, t]],
                                              rows_v.at[b], sems[b]).wait()
                        pltpu.sync_copy(rows_v.at[b],
                                        agg_sh.at[ei_v.at[1, t]], add=True)

                        @pl.when(kk < IDXH // NBUF - 1)
                        def _():
                            pltpu.async_copy(x_sh.at[ei_v.at[0, t + NBUF]],
                                             rows_v.at[b], sems[b])
            plsc.subcore_barrier()
            _writeback(sid, agg_sh, o_hbm)
            plsc.subcore_barrier()

        @pl.when(cid == 0)
        def _():
            one_pass(0, o0_hbm)
            one_pass(1, o1_hbm)

        @pl.when(cid == 1)
        def _():
            one_pass(2, o2_hbm)
            one_pass(3, o3_hbm)

    return k(x, ei3, zrows)


def _seg_sum_l2(p, ei3, zrows):
    mesh = plsc.VectorSubcoreMesh(core_axis_name="c", subcore_axis_name="s")
    tpt = NBLK // (NC * NS)

    @functools.partial(
        pl.kernel,
        mesh=mesh,
        compiler_params=pltpu.CompilerParams(use_tc_tiling_on_sc=False),
        out_type=[jax.ShapeDtypeStruct((N, P_W), jnp.float32),
                  jax.ShapeDtypeStruct((N, P_W), jnp.float32)],
        scratch_types=[
            pltpu.VMEM((2, NBLK // (NC * NS), CHUNK), jnp.int32),
            pltpu.VMEM((NBUF, CHUNK, P_W), jnp.float32),
            pltpu.VMEM_SHARED((N, P_W), jnp.float32),
            pltpu.VMEM_SHARED((NROW, P_W), jnp.float32),
        ] + [pltpu.SemaphoreType.DMA for _ in range(NBUF)],
    )
    def k(p_hbm, ei_hbm, z_hbm, oa_hbm, ob_hbm,
          ei_v, rows_v, p_sh, agg_sh, *sems):
        cid = lax.axis_index("c")
        sid = lax.axis_index("s")

        def run(lo_chunk, o_hbm):
            pltpu.sync_copy(
                ei_hbm.at[:, pl.ds(lo_chunk + sid * tpt, tpt), :], ei_v)

            @pl.loop(sid, NZCHUNK, step=NS)
            def _(j):
                pltpu.sync_copy(p_hbm.at[pl.ds(j * ZROWS, ZROWS)],
                                p_sh.at[pl.ds(j * ZROWS, ZROWS)])
                pltpu.sync_copy(z_hbm.at[pl.ds(0, ZROWS)],
                                agg_sh.at[pl.ds(j * ZROWS, ZROWS)])
            plsc.subcore_barrier()
            for b in range(NBUF):
                pltpu.async_copy(p_sh.at[ei_v.at[0, b]], rows_v.at[b],
                                 sems[b])

            @pl.loop(0, tpt // NBUF)
            def _(kk):
                for b in range(NBUF):
                    t = NBUF * kk + b
                    pltpu.make_async_copy(p_sh.at[ei_v.at[0, t]],
                                          rows_v.at[b], sems[b]).wait()
                    pltpu.sync_copy(rows_v.at[b],
                                    agg_sh.at[ei_v.at[1, t]], add=True)

                    @pl.when(kk < tpt // NBUF - 1)
                    def _():
                        pltpu.async_copy(p_sh.at[ei_v.at[0, t + NBUF]],
                                         rows_v.at[b], sems[b])
            plsc.subcore_barrier()
            _writeback(sid, agg_sh, o_hbm)

        @pl.when(cid == 0)
        def _():
            run(0, oa_hbm)

        @pl.when(cid == 1)
        def _():
            run(NBLK // NC, ob_hbm)

    return k(p, ei3, zrows)


_R = 1000


def _tc_a(x, w1_top, b1):
    def body(x_ref, w_ref, b_ref, o_ref):
        o_ref[...] = jnp.dot(x_ref[...], w_ref[...],
                             preferred_element_type=jnp.float32) + b_ref[...]

    return pl.pallas_call(
        body,
        grid=(N // _R,),
        in_specs=[pl.BlockSpec((_R, D_IN), lambda i: (i, 0)),
                  pl.BlockSpec((D_IN, D_HID), lambda i: (0, 0)),
                  pl.BlockSpec((1, D_HID), lambda i: (0, 0))],
        out_specs=pl.BlockSpec((_R, D_HID), lambda i: (i, 0)),
        out_shape=jax.ShapeDtypeStruct((N, D_HID), jnp.float32),
    )(x, w1_top, b1.reshape(1, D_HID))


def _tc_b(u, aggs, w1_bot, w2_top, w2_bot_pad, b2):
    def body(u_ref, a0_ref, a1_ref, a2_ref, a3_ref, wb_ref, wt_ref, wp_ref,
             b2_ref, t_ref, p_ref):
        agg = jnp.concatenate(
            [a0_ref[...], a1_ref[...], a2_ref[...], a3_ref[...]], axis=1)
        h = u_ref[...] + jnp.dot(agg, wb_ref[...],
                                 preferred_element_type=jnp.float32)
        h = jnp.maximum(h, 0.0)
        t_ref[...] = jnp.dot(h, wt_ref[...],
                             preferred_element_type=jnp.float32) + b2_ref[...]
        p_ref[...] = jnp.dot(h, wp_ref[...],
                             preferred_element_type=jnp.float32)

    return pl.pallas_call(
        body,
        grid=(N // _R,),
        in_specs=[pl.BlockSpec((_R, D_HID), lambda i: (i, 0))] +
                 [pl.BlockSpec((_R, QW), lambda i: (i, 0))
                  for _ in range(4)] +
                 [pl.BlockSpec((D_IN, D_HID), lambda i: (0, 0)),
                  pl.BlockSpec((D_HID, D_OUT), lambda i: (0, 0)),
                  pl.BlockSpec((D_HID, P_W), lambda i: (0, 0)),
                  pl.BlockSpec((1, D_OUT), lambda i: (0, 0))],
        out_specs=[pl.BlockSpec((_R, D_OUT), lambda i: (i, 0)),
                   pl.BlockSpec((_R, P_W), lambda i: (i, 0))],
        out_shape=[jax.ShapeDtypeStruct((N, D_OUT), jnp.float32),
                   jax.ShapeDtypeStruct((N, P_W), jnp.float32)],
    )(u, *aggs, w1_bot, w2_top, w2_bot_pad, b2.reshape(1, D_OUT))


def _tc_c(t, qa, qb):
    def body(t_ref, a_ref, b_ref, o_ref):
        o_ref[...] = t_ref[...] + a_ref[...] + b_ref[...]

    return pl.pallas_call(
        body,
        grid=(N // _R,),
        in_specs=[pl.BlockSpec((_R, D_OUT), lambda i: (i, 0)),
                  pl.BlockSpec((_R, D_OUT), lambda i: (i, 0)),
                  pl.BlockSpec((_R, D_OUT), lambda i: (i, 0))],
        out_specs=pl.BlockSpec((_R, D_OUT), lambda i: (i, 0)),
        out_shape=jax.ShapeDtypeStruct((N, D_OUT), jnp.float32),
    )(t, qa, qb)


def kernel(x, edge_index, W1, b1, W2, b2):
    src = jnp.concatenate([edge_index[0],
                           jnp.zeros((E_PAD - E,), jnp.int32)])
    dst = jnp.concatenate([edge_index[1],
                           jnp.full((E_PAD - E,), N, jnp.int32)])
    ei3 = jnp.stack([src, dst]).reshape(2, NBLK, CHUNK)
    z1 = jnp.zeros((ZROWS, QW), jnp.float32)
    z2 = jnp.zeros((ZROWS, P_W), jnp.float32)

    w1_top = W1[:D_IN]
    w1_bot = W1[D_IN:]
    w2_top = W2[:D_HID]
    w2_bot_pad = jnp.concatenate(
        [W2[D_HID:], jnp.zeros((D_HID, P_W - D_OUT), jnp.float32)], axis=1)

    aggs = _seg_sum_l1(x, ei3, z1)
    u = _tc_a(x, w1_top, b1)
    t, p = _tc_b(u, aggs, w1_bot, w2_top, w2_bot_pad, b2)
    pa, pb = _seg_sum_l2(p, ei3, z2)
    return _tc_c(t, pa[:, :D_OUT], pb[:, :D_OUT])
